# baseline (speedup 1.0000x reference)
import jax
import jax.numpy as jnp
from jax import lax
from jax.experimental import pallas as pl
from jax.experimental.pallas import tpu as pltpu
from jax.experimental.pallas import tpu_sc as plsc

N_NODES = 10000
N_EDGES = 320000
D = 128

NC = 2
NS = 16

E_PER_W = N_EDGES // NS
CHUNK = 80
N_CHUNKS = E_PER_W // CHUNK
SLOTS = 4
LOOKI = 8
STAGE_ROWS = 624
STAGE_TAIL = N_NODES - NS * STAGE_ROWS


def _sc_gather(m_hbm, root_hbm, idx_hbm, out_m, out_root,
               bufs, idxv, tab, sem_i, sem_g, sem_w):
    cid = lax.axis_index("c")
    sid = lax.axis_index("s")
    base = sid * E_PER_W

    stripe = pl.ds(pl.multiple_of(sid * STAGE_ROWS, 8), STAGE_ROWS)
    tail = pl.ds(NS * STAGE_ROWS, STAGE_TAIL)

    @pl.when(cid == 0)
    def _stage_m():
        pltpu.sync_copy(m_hbm.at[stripe], tab.at[stripe])

        @pl.when(sid == 0)
        def _tail():
            pltpu.sync_copy(m_hbm.at[tail], tab.at[tail])

    @pl.when(cid == 1)
    def _stage_root():
        pltpu.sync_copy(root_hbm.at[stripe], tab.at[stripe])

        @pl.when(sid == 0)
        def _tail():
            pltpu.sync_copy(root_hbm.at[tail], tab.at[tail])

    plsc.subcore_barrier()

    def run_pipe(out):
        for p in range(LOOKI):
            pltpu.make_async_copy(idx_hbm.at[sid, p], idxv.at[p],
                                  sem_i).start()

        def body(k, carry):
            slot = lax.rem(k, SLOTS)
            islot = lax.rem(k, LOOKI)

            @pl.when(k >= SLOTS)
            def _reclaim():
                pltpu.make_async_copy(
                    bufs.at[slot],
                    out.at[pl.ds(base + (k - SLOTS) * CHUNK, CHUNK)],
                    sem_w).wait()

            pltpu.make_async_copy(idx_hbm.at[sid, k], idxv.at[islot],
                                  sem_i).wait()
            gather = pltpu.make_async_copy(tab.at[idxv.at[islot]],
                                           bufs.at[slot], sem_g)
            gather.start()
            gather.wait()

            @pl.when(k + LOOKI < N_CHUNKS)
            def _prefetch():
                pltpu.make_async_copy(idx_hbm.at[sid, k + LOOKI],
                                      idxv.at[islot], sem_i).start()

            pltpu.make_async_copy(
                bufs.at[slot],
                out.at[pl.ds(base + k * CHUNK, CHUNK)],
                sem_w).start()
            return carry

        lax.fori_loop(0, N_CHUNKS, body, 0)

        for j in range(SLOTS):
            k = N_CHUNKS - SLOTS + j
            pltpu.make_async_copy(
                bufs.at[k % SLOTS],
                out.at[pl.ds(base + k * CHUNK, CHUNK)],
                sem_w).wait()

    @pl.when(cid == 0)
    def _produce_m():
        run_pipe(out_m)

    @pl.when(cid == 1)
    def _produce_root():
        run_pipe(out_root)


@jax.jit
def kernel(m, root, edge_index):
    src = edge_index[0].astype(jnp.int32).reshape(NS, N_CHUNKS, CHUNK)
    mesh = plsc.VectorSubcoreMesh(core_axis_name="c", subcore_axis_name="s")
    out_ty = (jax.ShapeDtypeStruct((N_EDGES, D), jnp.float32),
              jax.ShapeDtypeStruct((N_EDGES, D), jnp.float32))
    f = pl.kernel(
        _sc_gather,
        mesh=mesh,
        out_type=out_ty,
        scratch_types=[
            pltpu.VMEM((SLOTS, CHUNK, D), jnp.float32),
            pltpu.VMEM((LOOKI, CHUNK), jnp.int32),
            pltpu.VMEM_SHARED((N_NODES, D), jnp.float32),
            pltpu.SemaphoreType.DMA,
            pltpu.SemaphoreType.DMA,
            pltpu.SemaphoreType.DMA,
        ],
    )
    return f(m, root, src)


# speedup vs baseline: 1.1061x; 1.1061x over previous
import jax
import jax.numpy as jnp
from jax import lax
from jax.experimental import pallas as pl
from jax.experimental.pallas import tpu as pltpu
from jax.experimental.pallas import tpu_sc as plsc

N_NODES = 10000
N_EDGES = 320000
D = 128

NC = 2
NS = 16

E_PER_W = N_EDGES // NS
CHUNK = 80
N_CHUNKS = E_PER_W // CHUNK
SLOTS = 4
LOOKI = 8
STAGE_ROWS = 624
STAGE_TAIL = N_NODES - NS * STAGE_ROWS


def _sc_gather(m_hbm, root_hbm, idx_hbm, out_m, out_root,
               bufs, idxv, tab, sem_i, sem_g, sem_w):
    cid = lax.axis_index("c")
    sid = lax.axis_index("s")
    base = sid * E_PER_W

    stripe = pl.ds(pl.multiple_of(sid * STAGE_ROWS, 8), STAGE_ROWS)
    tail = pl.ds(NS * STAGE_ROWS, STAGE_TAIL)

    @pl.when(cid == 0)
    def _stage_m():
        pltpu.sync_copy(m_hbm.at[stripe], tab.at[stripe])

        @pl.when(sid == 0)
        def _tail():
            pltpu.sync_copy(m_hbm.at[tail], tab.at[tail])

    @pl.when(cid == 1)
    def _stage_root():
        pltpu.sync_copy(root_hbm.at[stripe], tab.at[stripe])

        @pl.when(sid == 0)
        def _tail():
            pltpu.sync_copy(root_hbm.at[tail], tab.at[tail])

    plsc.subcore_barrier()

    def run_pipe(out):
        for p in range(LOOKI):
            pltpu.make_async_copy(idx_hbm.at[sid, p], idxv.at[p],
                                  sem_i).start()

        def body(k, carry):
            slot = lax.rem(k, SLOTS)
            islot = lax.rem(k, LOOKI)

            @pl.when(k >= SLOTS)
            def _reclaim():
                pltpu.make_async_copy(
                    bufs.at[slot],
                    out.at[pl.ds(base + (k - SLOTS) * CHUNK, CHUNK)],
                    sem_w).wait()

            pltpu.make_async_copy(idx_hbm.at[sid, k], idxv.at[islot],
                                  sem_i).wait()
            pltpu.make_async_copy(tab.at[idxv.at[islot]],
                                  bufs.at[slot], sem_g).start()

            @pl.when(k >= 1)
            def _emit_prev():
                pslot = lax.rem(k - 1, SLOTS)
                pislot = lax.rem(k - 1, LOOKI)
                pltpu.make_async_copy(tab.at[idxv.at[pislot]],
                                      bufs.at[pslot], sem_g).wait()

                @pl.when(k - 1 + LOOKI < N_CHUNKS)
                def _prefetch():
                    pltpu.make_async_copy(idx_hbm.at[sid, k - 1 + LOOKI],
                                          idxv.at[pislot], sem_i).start()

                pltpu.make_async_copy(
                    bufs.at[pslot],
                    out.at[pl.ds(base + (k - 1) * CHUNK, CHUNK)],
                    sem_w).start()
            return carry

        lax.fori_loop(0, N_CHUNKS, body, 0)

        last = N_CHUNKS - 1
        pltpu.make_async_copy(tab.at[idxv.at[last % LOOKI]],
                              bufs.at[last % SLOTS], sem_g).wait()
        pltpu.make_async_copy(
            bufs.at[last % SLOTS],
            out.at[pl.ds(base + last * CHUNK, CHUNK)],
            sem_w).start()

        for j in range(SLOTS):
            k = N_CHUNKS - SLOTS + j
            pltpu.make_async_copy(
                bufs.at[k % SLOTS],
                out.at[pl.ds(base + k * CHUNK, CHUNK)],
                sem_w).wait()

    @pl.when(cid == 0)
    def _produce_m():
        run_pipe(out_m)

    @pl.when(cid == 1)
    def _produce_root():
        run_pipe(out_root)


@jax.jit
def kernel(m, root, edge_index):
    src = edge_index[0].astype(jnp.int32).reshape(NS, N_CHUNKS, CHUNK)
    mesh = plsc.VectorSubcoreMesh(core_axis_name="c", subcore_axis_name="s")
    out_ty = (jax.ShapeDtypeStruct((N_EDGES, D), jnp.float32),
              jax.ShapeDtypeStruct((N_EDGES, D), jnp.float32))
    f = pl.kernel(
        _sc_gather,
        mesh=mesh,
        out_type=out_ty,
        scratch_types=[
            pltpu.VMEM((SLOTS, CHUNK, D), jnp.float32),
            pltpu.VMEM((LOOKI, CHUNK), jnp.int32),
            pltpu.VMEM_SHARED((N_NODES, D), jnp.float32),
            pltpu.SemaphoreType.DMA,
            pltpu.SemaphoreType.DMA,
            pltpu.SemaphoreType.DMA,
        ],
    )
    return f(m, root, src)
